# baseline (speedup 1.0000x reference)
import functools

import jax
import jax.numpy as jnp
from jax import lax
from jax.experimental import pallas as pl
from jax.experimental.pallas import tpu as pltpu
from jax.experimental.pallas import tpu_sc as plsc

B = 2
N_COARSE = 25000
N_FINE = 100000
D = 128

NC = 2
NS = 16
NW = NC * NS

ROWS = B * N_FINE
CHUNK = 128
ROWS_PAD = 204800
WPW = ROWS_PAD // NW
CPW = WPW // CHUNK

TILE_A = 1000
TILE_C = 1000


def _matmul_a_body(x_ref, w2_ref, o_ref):
    o_ref[...] = jnp.dot(x_ref[...], w2_ref[...],
                         preferred_element_type=jnp.float32)


def _coarse_matmul(x_flat, w2):
    n = x_flat.shape[0]
    return pl.pallas_call(
        _matmul_a_body,
        grid=(n // TILE_A,),
        in_specs=[
            pl.BlockSpec((TILE_A, D), lambda i: (i, 0)),
            pl.BlockSpec((D, D), lambda i: (0, 0)),
        ],
        out_specs=pl.BlockSpec((TILE_A, D), lambda i: (i, 0)),
        out_shape=jax.ShapeDtypeStruct((n, D), jnp.float32),
    )(x_flat, w2)


def _gather_body(src_hbm, gidx_hbm, out_hbm, idx_v, rows_v, sem):
    wid = lax.axis_index("s") * NC + lax.axis_index("c")
    base = wid * WPW
    pltpu.sync_copy(gidx_hbm.at[pl.ds(base, WPW)], idx_v)

    def body(c, carry):
        off = pl.multiple_of(c * CHUNK, CHUNK)
        pltpu.async_copy(
            src_hbm.at[idx_v.at[pl.ds(off, CHUNK)]], rows_v, sem
        ).wait()
        pltpu.sync_copy(rows_v, out_hbm.at[pl.ds(base + off, CHUNK)])
        return carry

    lax.fori_loop(0, CPW, body, 0)


def _sc_gather(src, gidx):
    mesh = plsc.VectorSubcoreMesh(core_axis_name="c", subcore_axis_name="s")
    f = functools.partial(
        pl.kernel,
        mesh=mesh,
        out_type=jax.ShapeDtypeStruct((ROWS_PAD, D), jnp.float32),
        scratch_types=[
            pltpu.VMEM((WPW,), jnp.int32),
            pltpu.VMEM((CHUNK, D), jnp.float32),
            pltpu.SemaphoreType.DMA,
        ],
    )(_gather_body)
    return f(src, gidx)


def _final_body(xs_ref, g_ref, d_ref, w1_ref, b_ref, o_ref):
    w = 1.0 / (d_ref[...] + 1e-2)
    acc = jnp.dot(xs_ref[0], w1_ref[...], preferred_element_type=jnp.float32)
    o_ref[0] = acc + w * g_ref[...] + b_ref[...]


def _final(x_scale, g, dist2, w1, b2):
    return pl.pallas_call(
        _final_body,
        grid=(B, N_FINE // TILE_C),
        in_specs=[
            pl.BlockSpec((1, TILE_C, D), lambda bb, i: (bb, i, 0)),
            pl.BlockSpec((TILE_C, D), lambda bb, i: (bb * (N_FINE // TILE_C) + i, 0)),
            pl.BlockSpec((TILE_C, 1), lambda bb, i: (i, 0)),
            pl.BlockSpec((D, D), lambda bb, i: (0, 0)),
            pl.BlockSpec((1, D), lambda bb, i: (0, 0)),
        ],
        out_specs=pl.BlockSpec((1, TILE_C, D), lambda bb, i: (bb, i, 0)),
        out_shape=jax.ShapeDtypeStruct((B, N_FINE, D), jnp.float32),
    )(x_scale, g, dist2, w1, b2)


def kernel(x, x_scale, fine2coarse_index, distances, W, b):
    w1 = W[:D]
    w2 = W[D:]
    x_flat = x.reshape(B * N_COARSE, D)
    idx = fine2coarse_index.astype(jnp.int32)
    gidx = jnp.concatenate(
        [idx, idx + N_COARSE, jnp.zeros((ROWS_PAD - ROWS,), jnp.int32)]
    )
    dist2 = distances.reshape(N_FINE, 1)
    b2 = b.reshape(1, D)

    xw2 = _coarse_matmul(x_flat, w2)
    g = _sc_gather(xw2, gidx)
    return _final(x_scale, g, dist2, w1, b2)


# speedup vs baseline: 1.0637x; 1.0637x over previous
import functools

import jax
import jax.numpy as jnp
from jax import lax
from jax.experimental import pallas as pl
from jax.experimental.pallas import tpu as pltpu
from jax.experimental.pallas import tpu_sc as plsc

B = 2
N_COARSE = 25000
N_FINE = 100000
D = 128

NC = 2
NS = 16
NW = NC * NS

ROWS = B * N_FINE
CHUNK = 128
ROWS_PAD = 204800
WPW = ROWS_PAD // NW
CPW = WPW // CHUNK
NBUF = 5
NGROUP = CPW // NBUF

TILE_A = 1000
TILE_C = 1000


def _matmul_a_body(x_ref, w2_ref, o_ref):
    o_ref[...] = jnp.dot(x_ref[...], w2_ref[...],
                         preferred_element_type=jnp.float32)


def _coarse_matmul(x_flat, w2):
    n = x_flat.shape[0]
    return pl.pallas_call(
        _matmul_a_body,
        grid=(n // TILE_A,),
        in_specs=[
            pl.BlockSpec((TILE_A, D), lambda i: (i, 0)),
            pl.BlockSpec((D, D), lambda i: (0, 0)),
        ],
        out_specs=pl.BlockSpec((TILE_A, D), lambda i: (i, 0)),
        out_shape=jax.ShapeDtypeStruct((n, D), jnp.float32),
    )(x_flat, w2)


def _gather_body(src_hbm, gidx_hbm, out_hbm, idx_v, *rest):
    bufs = rest[:NBUF]
    gsems = rest[NBUF:2 * NBUF]
    wsems = rest[2 * NBUF:3 * NBUF]
    wid = lax.axis_index("s") * NC + lax.axis_index("c")
    base = wid * WPW
    pltpu.sync_copy(gidx_hbm.at[pl.ds(base, WPW)], idx_v)

    def group(g, carry):
        hs = []
        for bb in range(NBUF):
            off = pl.multiple_of((g * NBUF + bb) * CHUNK, CHUNK)
            hs.append(pltpu.async_copy(
                src_hbm.at[idx_v.at[pl.ds(off, CHUNK)]], bufs[bb], gsems[bb]))
        ws = []
        for bb in range(NBUF):
            off = pl.multiple_of((g * NBUF + bb) * CHUNK, CHUNK)
            hs[bb].wait()
            ws.append(pltpu.async_copy(
                bufs[bb], out_hbm.at[pl.ds(base + off, CHUNK)], wsems[bb]))
        for bb in range(NBUF):
            ws[bb].wait()
        return carry

    lax.fori_loop(0, NGROUP, group, 0)


def _sc_gather(src, gidx):
    mesh = plsc.VectorSubcoreMesh(core_axis_name="c", subcore_axis_name="s")
    f = functools.partial(
        pl.kernel,
        mesh=mesh,
        out_type=jax.ShapeDtypeStruct((ROWS_PAD, D), jnp.float32),
        scratch_types=(
            [pltpu.VMEM((WPW,), jnp.int32)]
            + [pltpu.VMEM((CHUNK, D), jnp.float32) for _ in range(NBUF)]
            + [pltpu.SemaphoreType.DMA for _ in range(2 * NBUF)]
        ),
    )(_gather_body)
    return f(src, gidx)


def _final_body(xs_ref, g_ref, d_ref, w1_ref, b_ref, o_ref):
    w = 1.0 / (d_ref[...] + 1e-2)
    acc = jnp.dot(xs_ref[0], w1_ref[...], preferred_element_type=jnp.float32)
    o_ref[0] = acc + w * g_ref[...] + b_ref[...]


def _final(x_scale, g, dist2, w1, b2):
    return pl.pallas_call(
        _final_body,
        grid=(B, N_FINE // TILE_C),
        in_specs=[
            pl.BlockSpec((1, TILE_C, D), lambda bb, i: (bb, i, 0)),
            pl.BlockSpec((TILE_C, D), lambda bb, i: (bb * (N_FINE // TILE_C) + i, 0)),
            pl.BlockSpec((TILE_C, 1), lambda bb, i: (i, 0)),
            pl.BlockSpec((D, D), lambda bb, i: (0, 0)),
            pl.BlockSpec((1, D), lambda bb, i: (0, 0)),
        ],
        out_specs=pl.BlockSpec((1, TILE_C, D), lambda bb, i: (bb, i, 0)),
        out_shape=jax.ShapeDtypeStruct((B, N_FINE, D), jnp.float32),
    )(x_scale, g, dist2, w1, b2)


def kernel(x, x_scale, fine2coarse_index, distances, W, b):
    w1 = W[:D]
    w2 = W[D:]
    x_flat = x.reshape(B * N_COARSE, D)
    idx = fine2coarse_index.astype(jnp.int32)
    gidx = jnp.concatenate(
        [idx, idx + N_COARSE, jnp.zeros((ROWS_PAD - ROWS,), jnp.int32)]
    )
    dist2 = distances.reshape(N_FINE, 1)
    b2 = b.reshape(1, D)

    xw2 = _coarse_matmul(x_flat, w2)
    g = _sc_gather(xw2, gidx)
    return _final(x_scale, g, dist2, w1, b2)


# speedup vs baseline: 1.1913x; 1.1200x over previous
import functools

import jax
import jax.numpy as jnp
from jax import lax
from jax.experimental import pallas as pl
from jax.experimental.pallas import tpu as pltpu
from jax.experimental.pallas import tpu_sc as plsc

B = 2
N_COARSE = 25000
N_FINE = 100000
D = 128
DW = D // 2

NC = 2
NS = 16
NW = NC * NS

ROWS = B * N_FINE
CHUNK = 128
ROWS_PAD = 204800
WPW = ROWS_PAD // NW
CPW = WPW // CHUNK
NBUF = 5
NGROUP = CPW // NBUF

TILE_A = 1000
TILE_C = 1000

HIMASK = -65536


def _matmul_a_body(x_ref, w2_ref, o_ref):
    xb = x_ref[...].astype(jnp.bfloat16)
    wb = w2_ref[...].astype(jnp.bfloat16)
    acc = jnp.dot(xb, wb, preferred_element_type=jnp.float32)
    lo = lax.bitcast_convert_type(acc[:, :DW], jnp.int32)
    hi = lax.bitcast_convert_type(acc[:, DW:], jnp.int32)
    o_ref[...] = lax.shift_right_logical(lo, 16) | (hi & HIMASK)


def _coarse_matmul(x_flat, w2):
    n = x_flat.shape[0]
    return pl.pallas_call(
        _matmul_a_body,
        grid=(n // TILE_A,),
        in_specs=[
            pl.BlockSpec((TILE_A, D), lambda i: (i, 0)),
            pl.BlockSpec((D, D), lambda i: (0, 0)),
        ],
        out_specs=pl.BlockSpec((TILE_A, DW), lambda i: (i, 0)),
        out_shape=jax.ShapeDtypeStruct((n, DW), jnp.int32),
    )(x_flat, w2)


def _gather_body(src_hbm, gidx_hbm, out_hbm, idx_v, *rest):
    bufs = rest[:NBUF]
    gsems = rest[NBUF:2 * NBUF]
    wsems = rest[2 * NBUF:3 * NBUF]
    wid = lax.axis_index("s") * NC + lax.axis_index("c")
    base = wid * WPW
    pltpu.sync_copy(gidx_hbm.at[pl.ds(base, WPW)], idx_v)

    def group(g, carry):
        hs = []
        for bb in range(NBUF):
            off = pl.multiple_of((g * NBUF + bb) * CHUNK, CHUNK)
            hs.append(pltpu.async_copy(
                src_hbm.at[idx_v.at[pl.ds(off, CHUNK)]], bufs[bb], gsems[bb]))
        ws = []
        for bb in range(NBUF):
            off = pl.multiple_of((g * NBUF + bb) * CHUNK, CHUNK)
            hs[bb].wait()
            ws.append(pltpu.async_copy(
                bufs[bb], out_hbm.at[pl.ds(base + off, CHUNK)], wsems[bb]))
        for bb in range(NBUF):
            ws[bb].wait()
        return carry

    lax.fori_loop(0, NGROUP, group, 0)


def _sc_gather(src, gidx):
    mesh = plsc.VectorSubcoreMesh(core_axis_name="c", subcore_axis_name="s")
    f = functools.partial(
        pl.kernel,
        mesh=mesh,
        compiler_params=pltpu.CompilerParams(use_tc_tiling_on_sc=False),
        out_type=jax.ShapeDtypeStruct((ROWS_PAD, DW), jnp.int32),
        scratch_types=(
            [pltpu.VMEM((WPW,), jnp.int32)]
            + [pltpu.VMEM((CHUNK, DW), jnp.int32) for _ in range(NBUF)]
            + [pltpu.SemaphoreType.DMA for _ in range(2 * NBUF)]
        ),
    )(_gather_body)
    return f(src, gidx)


def _final_body(xs_ref, g_ref, d_ref, w1_ref, b_ref, o_ref):
    w = 1.0 / (d_ref[...] + 1e-2)
    xb = xs_ref[0].astype(jnp.bfloat16)
    wb = w1_ref[...].astype(jnp.bfloat16)
    acc = jnp.dot(xb, wb, preferred_element_type=jnp.float32)
    gp = g_ref[...]
    glo = lax.bitcast_convert_type(lax.shift_left(gp, 16), jnp.float32)
    ghi = lax.bitcast_convert_type(gp & HIMASK, jnp.float32)
    gf = jnp.concatenate([glo, ghi], axis=1)
    o_ref[0] = acc + w * gf + b_ref[...]


def _final(x_scale, g, dist2, w1, b2):
    return pl.pallas_call(
        _final_body,
        grid=(B, N_FINE // TILE_C),
        in_specs=[
            pl.BlockSpec((1, TILE_C, D), lambda bb, i: (bb, i, 0)),
            pl.BlockSpec((TILE_C, DW), lambda bb, i: (bb * (N_FINE // TILE_C) + i, 0)),
            pl.BlockSpec((TILE_C, 1), lambda bb, i: (i, 0)),
            pl.BlockSpec((D, D), lambda bb, i: (0, 0)),
            pl.BlockSpec((1, D), lambda bb, i: (0, 0)),
        ],
        out_specs=pl.BlockSpec((1, TILE_C, D), lambda bb, i: (bb, i, 0)),
        out_shape=jax.ShapeDtypeStruct((B, N_FINE, D), jnp.float32),
    )(x_scale, g, dist2, w1, b2)


def kernel(x, x_scale, fine2coarse_index, distances, W, b):
    w1 = W[:D]
    w2 = W[D:]
    x_flat = x.reshape(B * N_COARSE, D)
    idx = fine2coarse_index.astype(jnp.int32)
    gidx = jnp.concatenate(
        [idx, idx + N_COARSE, jnp.zeros((ROWS_PAD - ROWS,), jnp.int32)]
    )
    dist2 = distances.reshape(N_FINE, 1)
    b2 = b.reshape(1, D)

    xw2p = _coarse_matmul(x_flat, w2)
    g = _sc_gather(xw2p, gidx)
    return _final(x_scale, g, dist2, w1, b2)


# speedup vs baseline: 1.3610x; 1.1424x over previous
import functools

import jax
import jax.numpy as jnp
from jax import lax
from jax.experimental import pallas as pl
from jax.experimental.pallas import tpu as pltpu
from jax.experimental.pallas import tpu_sc as plsc

B = 2
N_COARSE = 25000
N_FINE = 100000
D = 128
DW = D // 2

NC = 2
NS = 16
NW = NC * NS

ROWS = B * N_FINE
CHUNK = 128
ROWS_PAD = 204800
NBUF = 5
CPW0 = 75
CPW1 = 25
WPW0 = CPW0 * CHUNK
TOTAL_CHUNKS = NS * (CPW0 + CPW1)
GIDX_LEN = (NS * CPW0 + 15 * CPW1) * CHUNK + WPW0

TILE_A = 2000
TILE_C = 2000

HIMASK = -65536


def _matmul_a_body(x_ref, w2_ref, o_ref):
    xb = x_ref[...].astype(jnp.bfloat16)
    wb = w2_ref[...].astype(jnp.bfloat16)
    acc = jnp.dot(xb, wb, preferred_element_type=jnp.float32)
    lo = lax.bitcast_convert_type(acc[:, :DW], jnp.int32)
    hi = lax.bitcast_convert_type(acc[:, DW:], jnp.int32)
    o_ref[...] = lax.shift_right_logical(lo, 16) | (hi & HIMASK)


def _coarse_matmul(x_flat, w2):
    n = x_flat.shape[0]
    return pl.pallas_call(
        _matmul_a_body,
        grid=(n // TILE_A,),
        in_specs=[
            pl.BlockSpec((TILE_A, D), lambda i: (i, 0)),
            pl.BlockSpec((D, D), lambda i: (0, 0)),
        ],
        out_specs=pl.BlockSpec((TILE_A, DW), lambda i: (i, 0)),
        out_shape=jax.ShapeDtypeStruct((n, DW), jnp.int32),
    )(x_flat, w2)


def _gather_body(src_hbm, gidx_hbm, out_hbm, idx_v, *rest):
    bufs = rest[:NBUF]
    gsems = rest[NBUF:2 * NBUF]
    wsems = rest[2 * NBUF:3 * NBUF]
    cid = lax.axis_index("c")
    sid = lax.axis_index("s")
    base_chunk = lax.select(cid == 0, sid * CPW0, NS * CPW0 + sid * CPW1)
    my_rows = lax.select(cid == 0, CPW0 * CHUNK, CPW1 * CHUNK)
    base = base_chunk * CHUNK
    ngroups = lax.select(cid == 0, CPW0 // NBUF, CPW1 // NBUF)
    pltpu.sync_copy(gidx_hbm.at[pl.ds(base, WPW0)], idx_v)

    def group(g, carry):
        hs = []
        for bb in range(NBUF):
            off = pl.multiple_of((g * NBUF + bb) * CHUNK, CHUNK)
            hs.append(pltpu.async_copy(
                src_hbm.at[idx_v.at[pl.ds(off, CHUNK)]], bufs[bb], gsems[bb]))
        ws = []
        for bb in range(NBUF):
            off = pl.multiple_of((g * NBUF + bb) * CHUNK, CHUNK)
            hs[bb].wait()
            ws.append(pltpu.async_copy(
                bufs[bb], out_hbm.at[pl.ds(base + off, CHUNK)], wsems[bb]))
        for bb in range(NBUF):
            ws[bb].wait()
        return carry

    lax.fori_loop(0, ngroups, group, 0)


def _sc_gather(src, gidx):
    mesh = plsc.VectorSubcoreMesh(core_axis_name="c", subcore_axis_name="s")
    f = functools.partial(
        pl.kernel,
        mesh=mesh,
        compiler_params=pltpu.CompilerParams(use_tc_tiling_on_sc=False),
        out_type=jax.ShapeDtypeStruct((ROWS_PAD, DW), jnp.int32),
        scratch_types=(
            [pltpu.VMEM((WPW0,), jnp.int32)]
            + [pltpu.VMEM((CHUNK, DW), jnp.int32) for _ in range(NBUF)]
            + [pltpu.SemaphoreType.DMA for _ in range(2 * NBUF)]
        ),
    )(_gather_body)
    return f(src, gidx)


def _final_body(xs_ref, g_ref, d_ref, w1_ref, b_ref, o_ref):
    w = 1.0 / (d_ref[...] + 1e-2)
    xb = xs_ref[0].astype(jnp.bfloat16)
    wb = w1_ref[...].astype(jnp.bfloat16)
    acc = jnp.dot(xb, wb, preferred_element_type=jnp.float32)
    gp = g_ref[...]
    glo = lax.bitcast_convert_type(lax.shift_left(gp, 16), jnp.float32)
    ghi = lax.bitcast_convert_type(gp & HIMASK, jnp.float32)
    gf = jnp.concatenate([glo, ghi], axis=1)
    o_ref[0] = acc + w * gf + b_ref[...]


def _final(x_scale, g, dist2, w1, b2):
    return pl.pallas_call(
        _final_body,
        grid=(B, N_FINE // TILE_C),
        in_specs=[
            pl.BlockSpec((1, TILE_C, D), lambda bb, i: (bb, i, 0)),
            pl.BlockSpec((TILE_C, DW), lambda bb, i: (bb * (N_FINE // TILE_C) + i, 0)),
            pl.BlockSpec((TILE_C, 1), lambda bb, i: (i, 0)),
            pl.BlockSpec((D, D), lambda bb, i: (0, 0)),
            pl.BlockSpec((1, D), lambda bb, i: (0, 0)),
        ],
        out_specs=pl.BlockSpec((1, TILE_C, D), lambda bb, i: (bb, i, 0)),
        out_shape=jax.ShapeDtypeStruct((B, N_FINE, D), jnp.float32),
    )(x_scale, g, dist2, w1, b2)


def kernel(x, x_scale, fine2coarse_index, distances, W, b):
    w1 = W[:D]
    w2 = W[D:]
    x_flat = x.reshape(B * N_COARSE, D)
    idx = fine2coarse_index.astype(jnp.int32)
    gidx = jnp.concatenate(
        [idx, idx + N_COARSE, jnp.zeros((GIDX_LEN - ROWS,), jnp.int32)]
    )
    dist2 = distances.reshape(N_FINE, 1)
    b2 = b.reshape(1, D)

    xw2p = _coarse_matmul(x_flat, w2)
    g = _sc_gather(xw2p, gidx)
    return _final(x_scale, g, dist2, w1, b2)


# speedup vs baseline: 1.9428x; 1.4275x over previous
import functools

import jax
import jax.numpy as jnp
from jax import lax
from jax.experimental import pallas as pl
from jax.experimental.pallas import tpu as pltpu
from jax.experimental.pallas import tpu_sc as plsc

B = 2
N_COARSE = 25000
N_FINE = 100000
D = 128
DW = D // 2

NC = 2
NS = 16
NW = NC * NS

CHUNK = 128
NF_PAD = 102400
WPW = NF_PAD // NS
CPW = WPW // CHUNK
NBUF = 2
NGROUP = CPW // NBUF

TILE_A = 2000
TILE_C = 2000

HIMASK = -65536


def _matmul_a_body(x_ref, w2_ref, o_ref):
    xb = x_ref[...].astype(jnp.bfloat16)
    wb = w2_ref[...].astype(jnp.bfloat16)
    acc = jnp.dot(xb, wb, preferred_element_type=jnp.float32)
    lo = lax.bitcast_convert_type(acc[:, :DW], jnp.int32)
    hi = lax.bitcast_convert_type(acc[:, DW:], jnp.int32)
    o_ref[...] = lax.shift_right_logical(lo, 16) | (hi & HIMASK)


def _coarse_matmul(x_flat, w2):
    n = x_flat.shape[0]
    return pl.pallas_call(
        _matmul_a_body,
        grid=(n // TILE_A,),
        in_specs=[
            pl.BlockSpec((TILE_A, D), lambda i: (i, 0)),
            pl.BlockSpec((D, D), lambda i: (0, 0)),
        ],
        out_specs=pl.BlockSpec((TILE_A, DW), lambda i: (i, 0)),
        out_shape=jax.ShapeDtypeStruct((n, DW), jnp.int32),
    )(x_flat, w2)


def _gather_body(src_hbm, idx_hbm, out_hbm, table_s, idx_v, *rest):
    bufs = rest[:NBUF]
    gsems = rest[NBUF:2 * NBUF]
    wsems = rest[2 * NBUF:3 * NBUF]
    cid = lax.axis_index("c")
    sid = lax.axis_index("s")

    @pl.when(sid == 0)
    def _stage():
        pltpu.sync_copy(src_hbm.at[cid], table_s)

    plsc.subcore_barrier()

    base = sid * WPW
    pltpu.sync_copy(idx_hbm.at[pl.ds(base, WPW)], idx_v)

    def group(g, carry):
        hs = []
        for bb in range(NBUF):
            off = pl.multiple_of((g * NBUF + bb) * CHUNK, CHUNK)
            hs.append(pltpu.async_copy(
                table_s.at[idx_v.at[pl.ds(off, CHUNK)]], bufs[bb], gsems[bb]))
        ws = []
        for bb in range(NBUF):
            off = pl.multiple_of((g * NBUF + bb) * CHUNK, CHUNK)
            hs[bb].wait()
            ws.append(pltpu.async_copy(
                bufs[bb], out_hbm.at[cid, pl.ds(base + off, CHUNK)], wsems[bb]))
        for bb in range(NBUF):
            ws[bb].wait()
        return carry

    lax.fori_loop(0, NGROUP, group, 0)


def _sc_gather(src, idx_pad):
    mesh = plsc.VectorSubcoreMesh(core_axis_name="c", subcore_axis_name="s")
    f = functools.partial(
        pl.kernel,
        mesh=mesh,
        compiler_params=pltpu.CompilerParams(use_tc_tiling_on_sc=False),
        out_type=jax.ShapeDtypeStruct((B, NF_PAD, DW), jnp.int32),
        scratch_types=(
            [pltpu.VMEM_SHARED((N_COARSE, DW), jnp.int32)]
            + [pltpu.VMEM((WPW,), jnp.int32)]
            + [pltpu.VMEM((CHUNK, DW), jnp.int32) for _ in range(NBUF)]
            + [pltpu.SemaphoreType.DMA for _ in range(2 * NBUF)]
        ),
    )(_gather_body)
    return f(src, idx_pad)


def _final_body(xs_ref, g_ref, d_ref, w1_ref, b_ref, o_ref):
    w = 1.0 / (d_ref[...] + 1e-2)
    xb = xs_ref[0].astype(jnp.bfloat16)
    wb = w1_ref[...].astype(jnp.bfloat16)
    acc = jnp.dot(xb, wb, preferred_element_type=jnp.float32)
    gp = g_ref[0]
    glo = lax.bitcast_convert_type(lax.shift_left(gp, 16), jnp.float32)
    ghi = lax.bitcast_convert_type(gp & HIMASK, jnp.float32)
    gf = jnp.concatenate([glo, ghi], axis=1)
    o_ref[0] = acc + w * gf + b_ref[...]


def _final(x_scale, g, dist2, w1, b2):
    return pl.pallas_call(
        _final_body,
        grid=(B, N_FINE // TILE_C),
        in_specs=[
            pl.BlockSpec((1, TILE_C, D), lambda bb, i: (bb, i, 0)),
            pl.BlockSpec((1, TILE_C, DW), lambda bb, i: (bb, i, 0)),
            pl.BlockSpec((TILE_C, 1), lambda bb, i: (i, 0)),
            pl.BlockSpec((D, D), lambda bb, i: (0, 0)),
            pl.BlockSpec((1, D), lambda bb, i: (0, 0)),
        ],
        out_specs=pl.BlockSpec((1, TILE_C, D), lambda bb, i: (bb, i, 0)),
        out_shape=jax.ShapeDtypeStruct((B, N_FINE, D), jnp.float32),
    )(x_scale, g, dist2, w1, b2)


def kernel(x, x_scale, fine2coarse_index, distances, W, b):
    w1 = W[:D]
    w2 = W[D:]
    x_flat = x.reshape(B * N_COARSE, D)
    idx = fine2coarse_index.astype(jnp.int32)
    idx_pad = jnp.concatenate(
        [idx, jnp.zeros((NF_PAD - N_FINE,), jnp.int32)]
    )
    dist2 = distances.reshape(N_FINE, 1)
    b2 = b.reshape(1, D)

    xw2p = _coarse_matmul(x_flat, w2)
    src3 = xw2p.reshape(B, N_COARSE, DW)
    g = _sc_gather(src3, idx_pad)
    return _final(x_scale, g, dist2, w1, b2)


# speedup vs baseline: 2.2815x; 1.1744x over previous
import functools

import jax
import jax.numpy as jnp
from jax import lax
from jax.experimental import pallas as pl
from jax.experimental.pallas import tpu as pltpu
from jax.experimental.pallas import tpu_sc as plsc

B = 2
N_COARSE = 25000
N_FINE = 100000
D = 128
DW = D // 2

NC = 2
NS = 16
NW = NC * NS

CHUNK = 128
NF_PAD = 102400
WPW = NF_PAD // NS
CPW = WPW // CHUNK
NBUF = 2
NGROUP = CPW // NBUF

TILE_A = 2000
TILE_C = 2000

HIMASK = -65536


def _matmul_a_body(x_ref, w2_ref, o_ref):
    xb = x_ref[...].astype(jnp.bfloat16)
    wb = w2_ref[...].astype(jnp.bfloat16)
    acc = jnp.dot(xb, wb, preferred_element_type=jnp.float32)
    lo = lax.bitcast_convert_type(acc[:, :DW], jnp.int32)
    hi = lax.bitcast_convert_type(acc[:, DW:], jnp.int32)
    o_ref[...] = lax.shift_right_logical(lo, 16) | (hi & HIMASK)


def _coarse_matmul(x_flat, w2):
    n = x_flat.shape[0]
    return pl.pallas_call(
        _matmul_a_body,
        grid=(n // TILE_A,),
        in_specs=[
            pl.BlockSpec((TILE_A, D), lambda i: (i, 0)),
            pl.BlockSpec((D, D), lambda i: (0, 0)),
        ],
        out_specs=pl.BlockSpec((TILE_A, DW), lambda i: (i, 0)),
        out_shape=jax.ShapeDtypeStruct((n, DW), jnp.int32),
    )(x_flat, w2)


def _gather_body(src_hbm, idx_hbm, out_hbm, table_s, idx_v, *rest):
    bufs = rest[:NBUF]
    gsems = rest[NBUF:2 * NBUF]
    wsems = rest[2 * NBUF:3 * NBUF]
    cid = lax.axis_index("c")
    sid = lax.axis_index("s")

    @pl.when(sid == 0)
    def _stage():
        pltpu.sync_copy(src_hbm.at[cid], table_s)

    plsc.subcore_barrier()

    base = sid * WPW
    pltpu.sync_copy(idx_hbm.at[pl.ds(base, WPW)], idx_v)

    def group(g, carry):
        hs = []
        for bb in range(NBUF):
            off = pl.multiple_of((g * NBUF + bb) * CHUNK, CHUNK)
            hs.append(pltpu.async_copy(
                table_s.at[idx_v.at[pl.ds(off, CHUNK)]], bufs[bb], gsems[bb]))
        ws = []
        for bb in range(NBUF):
            off = pl.multiple_of((g * NBUF + bb) * CHUNK, CHUNK)
            hs[bb].wait()
            ws.append(pltpu.async_copy(
                bufs[bb], out_hbm.at[cid, pl.ds(base + off, CHUNK)], wsems[bb]))
        for bb in range(NBUF):
            ws[bb].wait()
        return carry

    lax.fori_loop(0, NGROUP, group, 0)


def _sc_gather(src, idx_pad):
    mesh = plsc.VectorSubcoreMesh(core_axis_name="c", subcore_axis_name="s")
    f = functools.partial(
        pl.kernel,
        mesh=mesh,
        compiler_params=pltpu.CompilerParams(use_tc_tiling_on_sc=False),
        out_type=jax.ShapeDtypeStruct((B, NF_PAD, DW), jnp.int32),
        scratch_types=(
            [pltpu.VMEM_SHARED((N_COARSE, DW), jnp.int32)]
            + [pltpu.VMEM((WPW,), jnp.int32)]
            + [pltpu.VMEM((CHUNK, DW), jnp.int32) for _ in range(NBUF)]
            + [pltpu.SemaphoreType.DMA for _ in range(2 * NBUF)]
        ),
    )(_gather_body)
    return f(src, idx_pad)


def _unpack16(gp):
    lo = lax.bitcast_convert_type(lax.shift_left(gp, 16), jnp.float32)
    hi = lax.bitcast_convert_type(gp & HIMASK, jnp.float32)
    return jnp.concatenate([lo, hi], axis=1)


def _final_body(xs_ref, g_ref, d_ref, w1_ref, b_ref, o_ref):
    w = 1.0 / (d_ref[...] + 1e-2)
    xb = xs_ref[0].astype(jnp.bfloat16)
    wb = w1_ref[...].astype(jnp.bfloat16)
    acc = jnp.dot(xb, wb, preferred_element_type=jnp.float32)
    gp = g_ref[0]
    even = _unpack16(gp[:, :DW])
    odd = _unpack16(gp[:, DW:])
    gf = jnp.concatenate(
        [even[:, None, :], odd[:, None, :]], axis=1
    ).reshape(TILE_C, D)
    o_ref[0] = acc + w * gf + b_ref[...]


def _final(x_scale, g3, dist2, w1, b2):
    return pl.pallas_call(
        _final_body,
        grid=(B, N_FINE // TILE_C),
        in_specs=[
            pl.BlockSpec((1, TILE_C, D), lambda bb, i: (bb, i, 0)),
            pl.BlockSpec((1, TILE_C // 2, D), lambda bb, i: (bb, i, 0)),
            pl.BlockSpec((TILE_C, 1), lambda bb, i: (i, 0)),
            pl.BlockSpec((D, D), lambda bb, i: (0, 0)),
            pl.BlockSpec((1, D), lambda bb, i: (0, 0)),
        ],
        out_specs=pl.BlockSpec((1, TILE_C, D), lambda bb, i: (bb, i, 0)),
        out_shape=jax.ShapeDtypeStruct((B, N_FINE, D), jnp.float32),
    )(x_scale, g3, dist2, w1, b2)


def kernel(x, x_scale, fine2coarse_index, distances, W, b):
    w1 = W[:D]
    w2 = W[D:]
    x_flat = x.reshape(B * N_COARSE, D)
    idx = fine2coarse_index.astype(jnp.int32)
    idx_pad = jnp.concatenate(
        [idx, jnp.zeros((NF_PAD - N_FINE,), jnp.int32)]
    )
    dist2 = distances.reshape(N_FINE, 1)
    b2 = b.reshape(1, D)

    xw2p = _coarse_matmul(x_flat, w2)
    src3 = xw2p.reshape(B, N_COARSE, DW)
    g = _sc_gather(src3, idx_pad)
    g3 = g.reshape(B, NF_PAD // 2, D)
    return _final(x_scale, g3, dist2, w1, b2)


# speedup vs baseline: 2.3152x; 1.0148x over previous
import functools

import jax
import jax.numpy as jnp
from jax import lax
from jax.experimental import pallas as pl
from jax.experimental.pallas import tpu as pltpu
from jax.experimental.pallas import tpu_sc as plsc

B = 2
N_COARSE = 25000
N_FINE = 100000
D = 128
DW = D // 2

NC = 2
NS = 16
NW = NC * NS

CHUNK = 128
NF_PAD = 102400
WPW = NF_PAD // NS
CPW = WPW // CHUNK
NBUF = 2
NGROUP = CPW // NBUF

TILE_A = 2000
TILE_C = 2000

HIMASK = -65536


def _matmul_a_body(x_ref, w2_ref, o_ref):
    xb = x_ref[...].astype(jnp.bfloat16)
    wb = w2_ref[...].astype(jnp.bfloat16)
    acc = jnp.dot(xb, wb, preferred_element_type=jnp.float32)
    lo = lax.bitcast_convert_type(acc[:, :DW], jnp.int32)
    hi = lax.bitcast_convert_type(acc[:, DW:], jnp.int32)
    o_ref[...] = lax.shift_right_logical(lo, 16) | (hi & HIMASK)


def _coarse_matmul(x_flat, w2):
    n = x_flat.shape[0]
    return pl.pallas_call(
        _matmul_a_body,
        grid=(n // TILE_A,),
        in_specs=[
            pl.BlockSpec((TILE_A, D), lambda i: (i, 0)),
            pl.BlockSpec((D, D), lambda i: (0, 0)),
        ],
        out_specs=pl.BlockSpec((TILE_A, DW), lambda i: (i, 0)),
        out_shape=jax.ShapeDtypeStruct((n, DW), jnp.int32),
    )(x_flat, w2)


def _gather_body(src_hbm, idx_hbm, out_hbm, table_s, idx_v, *rest):
    bufs = rest[:NBUF]
    gsems = rest[NBUF:2 * NBUF]
    wsems = rest[2 * NBUF:3 * NBUF]
    cid = lax.axis_index("c")
    sid = lax.axis_index("s")

    @pl.when(sid == 0)
    def _stage():
        pltpu.sync_copy(src_hbm.at[cid], table_s)

    plsc.subcore_barrier()

    base = sid * WPW
    pltpu.sync_copy(idx_hbm.at[pl.ds(base, WPW)], idx_v)

    def group(g, carry):
        hs = []
        for bb in range(NBUF):
            off = pl.multiple_of((g * NBUF + bb) * CHUNK, CHUNK)
            hs.append(pltpu.async_copy(
                table_s.at[idx_v.at[pl.ds(off, CHUNK)]], bufs[bb], gsems[bb]))
        ws = []
        for bb in range(NBUF):
            off = pl.multiple_of((g * NBUF + bb) * CHUNK, CHUNK)
            hs[bb].wait()
            ws.append(pltpu.async_copy(
                bufs[bb], out_hbm.at[cid, pl.ds(base + off, CHUNK)], wsems[bb]))
        for bb in range(NBUF):
            ws[bb].wait()
        return carry

    lax.fori_loop(0, NGROUP, group, 0)


def _sc_gather(src, idx_pad):
    mesh = plsc.VectorSubcoreMesh(core_axis_name="c", subcore_axis_name="s")
    f = functools.partial(
        pl.kernel,
        mesh=mesh,
        compiler_params=pltpu.CompilerParams(use_tc_tiling_on_sc=False),
        out_type=jax.ShapeDtypeStruct((B, NF_PAD, DW), jnp.int32),
        scratch_types=(
            [pltpu.VMEM_SHARED((N_COARSE, DW), jnp.int32)]
            + [pltpu.VMEM((WPW,), jnp.int32)]
            + [pltpu.VMEM((CHUNK, DW), jnp.int32) for _ in range(NBUF)]
            + [pltpu.SemaphoreType.DMA for _ in range(2 * NBUF)]
        ),
    )(_gather_body)
    return f(src, idx_pad)


def _unpack16(gp):
    lo = lax.bitcast_convert_type(lax.shift_left(gp, 16), jnp.float32)
    hi = lax.bitcast_convert_type(gp & HIMASK, jnp.float32)
    return jnp.concatenate([lo, hi], axis=1)


def _final_body(xs_ref, g_ref, d_ref, w1_ref, b_ref, o_ref):
    w = 1.0 / (d_ref[...] + 1e-2)
    xb = xs_ref[0].astype(jnp.bfloat16)
    wb = w1_ref[...].astype(jnp.bfloat16)
    acc = jnp.dot(xb, wb, preferred_element_type=jnp.float32)
    gp = g_ref[0]
    first = _unpack16(gp[:, :DW])
    second = _unpack16(gp[:, DW:])
    gf = jnp.concatenate([first, second], axis=0)
    o_ref[0] = acc + w * gf + b_ref[...]


def _final(x_scale, g3, dist2, w1, b2):
    return pl.pallas_call(
        _final_body,
        grid=(B, N_FINE // TILE_C),
        in_specs=[
            pl.BlockSpec((1, TILE_C, D), lambda bb, i: (bb, i, 0)),
            pl.BlockSpec((1, TILE_C // 2, D), lambda bb, i: (bb, i, 0)),
            pl.BlockSpec((TILE_C, 1), lambda bb, i: (i, 0)),
            pl.BlockSpec((D, D), lambda bb, i: (0, 0)),
            pl.BlockSpec((1, D), lambda bb, i: (0, 0)),
        ],
        out_specs=pl.BlockSpec((1, TILE_C, D), lambda bb, i: (bb, i, 0)),
        out_shape=jax.ShapeDtypeStruct((B, N_FINE, D), jnp.float32),
    )(x_scale, g3, dist2, w1, b2)


def kernel(x, x_scale, fine2coarse_index, distances, W, b):
    w1 = W[:D]
    w2 = W[D:]
    x_flat = x.reshape(B * N_COARSE, D)
    idx = fine2coarse_index.astype(jnp.int32)
    idxg = (idx.reshape(N_FINE // TILE_C, 2, TILE_C // 2)
            .transpose(0, 2, 1).reshape(N_FINE))
    idx_pad = jnp.concatenate(
        [idxg, jnp.zeros((NF_PAD - N_FINE,), jnp.int32)]
    )
    dist2 = distances.reshape(N_FINE, 1)
    b2 = b.reshape(1, D)

    xw2p = _coarse_matmul(x_flat, w2)
    src3 = xw2p.reshape(B, N_COARSE, DW)
    g = _sc_gather(src3, idx_pad)
    g3 = g.reshape(B, NF_PAD // 2, D)
    return _final(x_scale, g3, dist2, w1, b2)
